# baseline (speedup 1.0000x reference)
import jax
import jax.numpy as jnp
from jax.experimental import pallas as pl
from jax.experimental.pallas import tpu as pltpu

LAMBD_ = 0.5
CPAD = 1024
SENTINEL = 1.0e5
BN = 2048
NSUB = 4


def _loss_body(e_ref, t_ref, c_ref, col_ref, out_ref, ct_s, ct2b_s, c2_s,
               tot_s, cnt_s):
    i = pl.program_id(0)
    nsteps = pl.num_programs(0)
    c = c_ref.shape[0]

    @pl.when(i == 0)
    def _init():
        ct_s[...] = jnp.full(ct_s.shape, SENTINEL, jnp.float32)
        ct_s[:, :c] = c_ref[...].T
        ct0 = ct_s[...]
        c2_s[...] = jnp.sum(ct0 * ct0, axis=0, keepdims=True)
        ct2b_s[...] = (2.0 * ct0).astype(jnp.bfloat16)
        tot_s[...] = jnp.zeros_like(tot_s)
        cnt_s[...] = jnp.zeros_like(cnt_s)

    col = col_ref[...]
    c2 = c2_s[...]
    ct2b = ct2b_s[...]

    tots = []
    cnts = []
    sub = BN // NSUB
    for j in range(NSUB):
        rows = pl.ds(j * sub, sub)
        e = e_ref[rows, :]
        tgt = t_ref[rows, :]
        dot2 = jnp.dot(e.astype(jnp.bfloat16), ct2b,
                       preferred_element_type=jnp.float32)
        u = c2 - dot2
        onehot = col == tgt
        uat = jnp.sum(jnp.where(onehot, u, 0.0), axis=1, keepdims=True)
        diff = (LAMBD_ + uat) - u
        pos = diff > 0.0
        tots.append(jnp.sum(jnp.where(pos, diff, 0.0)))
        cnts.append(jnp.sum(pos.astype(jnp.float32)))

    tot_s[...] += sum(tots).reshape(1, 1)
    cnt_s[...] += sum(cnts).reshape(1, 1)

    @pl.when(i == nsteps - 1)
    def _fin():
        n = e_ref.shape[0] * nsteps
        total = tot_s[0, 0] - n * LAMBD_
        count = cnt_s[0, 0] - n
        loss = jnp.where(count > 0, total / jnp.maximum(count, 1.0), 0.0)
        out_ref[0] = loss


def kernel(embeddings, targets, centers):
    n, d = embeddings.shape
    c = centers.shape[0]
    tgt = targets.astype(jnp.int32).reshape(n, 1)
    col = jax.lax.iota(jnp.int32, CPAD).reshape(1, CPAD)

    out = pl.pallas_call(
        _loss_body,
        grid=(n // BN,),
        in_specs=[
            pl.BlockSpec((BN, d), lambda i: (i, 0)),
            pl.BlockSpec((BN, 1), lambda i: (i, 0)),
            pl.BlockSpec((c, d), lambda i: (0, 0)),
            pl.BlockSpec((1, CPAD), lambda i: (0, 0)),
        ],
        out_specs=pl.BlockSpec(memory_space=pltpu.SMEM),
        out_shape=jax.ShapeDtypeStruct((1,), jnp.float32),
        scratch_shapes=[
            pltpu.VMEM((d, CPAD), jnp.float32),
            pltpu.VMEM((d, CPAD), jnp.bfloat16),
            pltpu.VMEM((1, CPAD), jnp.float32),
            pltpu.VMEM((1, 1), jnp.float32),
            pltpu.VMEM((1, 1), jnp.float32),
        ],
    )(embeddings, tgt, centers, col)

    return out[0]


# speedup vs baseline: 1.0126x; 1.0126x over previous
import jax
import jax.numpy as jnp
from jax.experimental import pallas as pl
from jax.experimental.pallas import tpu as pltpu

LAMBD_ = 0.5
CPAD = 1024
SENTINEL = 1.0e5
BN = 2048
NSUB = 2


def _loss_body(e_ref, t_ref, c_ref, col_ref, out_ref, ct_s, ct2b_s, c2_s,
               tot_s, cnt_s):
    i = pl.program_id(0)
    nsteps = pl.num_programs(0)
    c = c_ref.shape[0]

    @pl.when(i == 0)
    def _init():
        ct_s[...] = jnp.full(ct_s.shape, SENTINEL, jnp.float32)
        ct_s[:, :c] = c_ref[...].T
        ct0 = ct_s[...]
        c2_s[...] = jnp.sum(ct0 * ct0, axis=0, keepdims=True)
        ct2b_s[...] = (2.0 * ct0).astype(jnp.bfloat16)
        tot_s[...] = jnp.zeros_like(tot_s)
        cnt_s[...] = jnp.zeros_like(cnt_s)

    col = col_ref[...]
    c2 = c2_s[...]
    ct2b = ct2b_s[...]

    tots = []
    cnts = []
    sub = BN // NSUB
    for j in range(NSUB):
        rows = pl.ds(j * sub, sub)
        e = e_ref[rows, :]
        tgt = t_ref[rows, :]
        dot2 = jnp.dot(e.astype(jnp.bfloat16), ct2b,
                       preferred_element_type=jnp.float32)
        u = c2 - dot2
        onehot = col == tgt
        uat = jnp.sum(jnp.where(onehot, u, 0.0), axis=1, keepdims=True)
        diff = (LAMBD_ + uat) - u
        pos = diff > 0.0
        tots.append(jnp.sum(jnp.where(pos, diff, 0.0)))
        cnts.append(jnp.sum(pos.astype(jnp.float32)))

    tot_s[...] += sum(tots).reshape(1, 1)
    cnt_s[...] += sum(cnts).reshape(1, 1)

    @pl.when(i == nsteps - 1)
    def _fin():
        n = e_ref.shape[0] * nsteps
        total = tot_s[0, 0] - n * LAMBD_
        count = cnt_s[0, 0] - n
        loss = jnp.where(count > 0, total / jnp.maximum(count, 1.0), 0.0)
        out_ref[0] = loss


def kernel(embeddings, targets, centers):
    n, d = embeddings.shape
    c = centers.shape[0]
    tgt = targets.astype(jnp.int32).reshape(n, 1)
    col = jax.lax.iota(jnp.int32, CPAD).reshape(1, CPAD)

    out = pl.pallas_call(
        _loss_body,
        grid=(n // BN,),
        in_specs=[
            pl.BlockSpec((BN, d), lambda i: (i, 0)),
            pl.BlockSpec((BN, 1), lambda i: (i, 0)),
            pl.BlockSpec((c, d), lambda i: (0, 0)),
            pl.BlockSpec((1, CPAD), lambda i: (0, 0)),
        ],
        out_specs=pl.BlockSpec(memory_space=pltpu.SMEM),
        out_shape=jax.ShapeDtypeStruct((1,), jnp.float32),
        scratch_shapes=[
            pltpu.VMEM((d, CPAD), jnp.float32),
            pltpu.VMEM((d, CPAD), jnp.bfloat16),
            pltpu.VMEM((1, CPAD), jnp.float32),
            pltpu.VMEM((1, 1), jnp.float32),
            pltpu.VMEM((1, 1), jnp.float32),
        ],
    )(embeddings, tgt, centers, col)

    return out[0]
